# baseline (device time: 318890 ns/iter reference)
import jax
import jax.numpy as jnp
from jax import lax
from jax.experimental import pallas as pl
from jax.experimental.pallas import tpu as pltpu

N_DEV = 4
CM = 512


def kernel(A, B):
    m, k = A.shape
    k2, n = B.shape

    def body(a_ref, b_ref, out_ref, comm_ref, send_sems, recv_sems):
        my = lax.axis_index("i")
        left = lax.rem(my - 1 + N_DEV, N_DEV)
        right = lax.rem(my + 1, N_DEV)

        barrier_sem = pltpu.get_barrier_semaphore()
        for nbr in (left, right):
            pl.semaphore_signal(
                barrier_sem, inc=1,
                device_id=(nbr,), device_id_type=pl.DeviceIdType.MESH,
            )
        pl.semaphore_wait(barrier_sem, 2)

        out_ref[:, :] = jnp.dot(
            a_ref[:, :], b_ref[:, :], preferred_element_type=jnp.float32
        )

        def chunk(ref, idx):
            return ref.at[pl.ds(idx * CM, CM), :]

        for h in range(N_DEV - 1):
            sc = lax.rem(my - h - 1 + N_DEV, N_DEV)
            rc = lax.rem(my - h - 2 + 2 * N_DEV, N_DEV)
            rdma = pltpu.make_async_remote_copy(
                src_ref=chunk(out_ref, sc),
                dst_ref=comm_ref.at[h],
                send_sem=send_sems.at[h],
                recv_sem=recv_sems.at[h],
                device_id=(right,),
                device_id_type=pl.DeviceIdType.MESH,
            )
            rdma.start()
            rdma.wait()
            out_ref[pl.ds(rc * CM, CM), :] = (
                out_ref[pl.ds(rc * CM, CM), :] + comm_ref[h]
            )

        out_ref[pl.ds(my * CM, CM), :] = jnp.maximum(
            out_ref[pl.ds(my * CM, CM), :], 0.0
        )

        for h in range(N_DEV - 1):
            ac = lax.rem(my - h + N_DEV, N_DEV)
            rdma = pltpu.make_async_remote_copy(
                src_ref=chunk(out_ref, ac),
                dst_ref=chunk(out_ref, ac),
                send_sem=send_sems.at[N_DEV - 1 + h],
                recv_sem=recv_sems.at[N_DEV - 1 + h],
                device_id=(right,),
                device_id_type=pl.DeviceIdType.MESH,
            )
            rdma.start()
            rdma.wait()

    return pl.pallas_call(
        body,
        out_shape=jax.ShapeDtypeStruct((m, n), jnp.float32),
        in_specs=[
            pl.BlockSpec(memory_space=pltpu.VMEM),
            pl.BlockSpec(memory_space=pltpu.VMEM),
        ],
        out_specs=pl.BlockSpec(memory_space=pltpu.VMEM),
        scratch_shapes=[
            pltpu.VMEM((N_DEV - 1, CM, n), jnp.float32),
            pltpu.SemaphoreType.DMA((2 * (N_DEV - 1),)),
            pltpu.SemaphoreType.DMA((2 * (N_DEV - 1),)),
        ],
        compiler_params=pltpu.CompilerParams(collective_id=0),
    )(A, B)


# device time: 177345 ns/iter; 1.7981x vs baseline; 1.7981x over previous
import jax
import jax.numpy as jnp
from jax import lax
from jax.experimental import pallas as pl
from jax.experimental.pallas import tpu as pltpu

N_DEV = 4
CM = 512
HL = 1024


def kernel(A, B):
    m, k = A.shape
    _, n = B.shape

    def body(a_ref, b_ref, out_ref,
             cw_comm, ccw_comm, cw_ssem, cw_rsem, ccw_ssem, ccw_rsem):
        my = lax.axis_index("i")
        left = lax.rem(my + N_DEV - 1, N_DEV)
        right = lax.rem(my + 1, N_DEV)

        barrier_sem = pltpu.get_barrier_semaphore()
        for nbr in (left, right):
            pl.semaphore_signal(
                barrier_sem, inc=1,
                device_id=(nbr,), device_id_type=pl.DeviceIdType.MESH,
            )
        pl.semaphore_wait(barrier_sem, 2)

        def blk(c, half):
            out_ref[pl.ds(c * CM, CM), pl.ds(half * HL, HL)] = jnp.dot(
                a_ref[pl.ds(c * CM, CM), :],
                b_ref[:, pl.ds(half * HL, HL)],
                preferred_element_type=jnp.float32,
            )

        def chunk(c, half):
            return out_ref.at[pl.ds(c * CM, CM), pl.ds(half * HL, HL)]

        def start_rs(h):
            cw = pltpu.make_async_remote_copy(
                src_ref=chunk(lax.rem(my + 3 - h, N_DEV), 0),
                dst_ref=cw_comm.at[h],
                send_sem=cw_ssem.at[h], recv_sem=cw_rsem.at[h],
                device_id=(right,), device_id_type=pl.DeviceIdType.MESH,
            )
            ccw = pltpu.make_async_remote_copy(
                src_ref=chunk(lax.rem(my + 1 + h, N_DEV), 1),
                dst_ref=ccw_comm.at[h],
                send_sem=ccw_ssem.at[h], recv_sem=ccw_rsem.at[h],
                device_id=(left,), device_id_type=pl.DeviceIdType.MESH,
            )
            cw.start()
            ccw.start()
            return cw, ccw

        blk(lax.rem(my + 3, N_DEV), 0)
        blk(lax.rem(my + 1, N_DEV), 1)
        inflight = start_rs(0)

        for h in range(N_DEV - 1):
            rc_cw = lax.rem(my + 2 - h + N_DEV, N_DEV)
            rc_ccw = lax.rem(my + 2 + h, N_DEV)
            blk(rc_cw, 0)
            blk(rc_ccw, 1)
            cw, ccw = inflight
            cw.wait()
            ccw.wait()
            out_ref[pl.ds(rc_cw * CM, CM), pl.ds(0, HL)] = (
                out_ref[pl.ds(rc_cw * CM, CM), pl.ds(0, HL)] + cw_comm[h]
            )
            out_ref[pl.ds(rc_ccw * CM, CM), pl.ds(HL, HL)] = (
                out_ref[pl.ds(rc_ccw * CM, CM), pl.ds(HL, HL)] + ccw_comm[h]
            )
            if h < N_DEV - 2:
                inflight = start_rs(h + 1)

        out_ref[pl.ds(my * CM, CM), :] = jnp.maximum(
            out_ref[pl.ds(my * CM, CM), :], 0.0
        )

        for h in range(N_DEV - 1):
            ac_cw = lax.rem(my + N_DEV - h, N_DEV)
            ac_ccw = lax.rem(my + h, N_DEV)
            cw = pltpu.make_async_remote_copy(
                src_ref=chunk(ac_cw, 0), dst_ref=chunk(ac_cw, 0),
                send_sem=cw_ssem.at[N_DEV - 1 + h],
                recv_sem=cw_rsem.at[N_DEV - 1 + h],
                device_id=(right,), device_id_type=pl.DeviceIdType.MESH,
            )
            ccw = pltpu.make_async_remote_copy(
                src_ref=chunk(ac_ccw, 1), dst_ref=chunk(ac_ccw, 1),
                send_sem=ccw_ssem.at[N_DEV - 1 + h],
                recv_sem=ccw_rsem.at[N_DEV - 1 + h],
                device_id=(left,), device_id_type=pl.DeviceIdType.MESH,
            )
            cw.start()
            ccw.start()
            cw.wait()
            ccw.wait()

    n_sems = 2 * (N_DEV - 1)
    return pl.pallas_call(
        body,
        out_shape=jax.ShapeDtypeStruct((m, n), jnp.float32),
        in_specs=[
            pl.BlockSpec(memory_space=pltpu.VMEM),
            pl.BlockSpec(memory_space=pltpu.VMEM),
        ],
        out_specs=pl.BlockSpec(memory_space=pltpu.VMEM),
        scratch_shapes=[
            pltpu.VMEM((N_DEV - 1, CM, HL), jnp.float32),
            pltpu.VMEM((N_DEV - 1, CM, HL), jnp.float32),
            pltpu.SemaphoreType.DMA((n_sems,)),
            pltpu.SemaphoreType.DMA((n_sems,)),
            pltpu.SemaphoreType.DMA((n_sems,)),
            pltpu.SemaphoreType.DMA((n_sems,)),
        ],
        compiler_params=pltpu.CompilerParams(collective_id=0),
    )(A, B)


# device time: 167163 ns/iter; 1.9077x vs baseline; 1.0609x over previous
import jax
import jax.numpy as jnp
from jax import lax
from jax.experimental import pallas as pl
from jax.experimental.pallas import tpu as pltpu

N_DEV = 4
CM = 512
HL = 1024
NSUB = 2
SUB = CM // NSUB


def kernel(A, B):
    m, k = A.shape
    _, n = B.shape

    def body(a_ref, b_ref, out_ref,
             cw_comm, ccw_comm, cw_ssem, cw_rsem, ccw_ssem, ccw_rsem):
        my = lax.axis_index("i")
        left = lax.rem(my + N_DEV - 1, N_DEV)
        right = lax.rem(my + 1, N_DEV)

        barrier_sem = pltpu.get_barrier_semaphore()
        for nbr in (left, right):
            pl.semaphore_signal(
                barrier_sem, inc=1,
                device_id=(nbr,), device_id_type=pl.DeviceIdType.MESH,
            )
        pl.semaphore_wait(barrier_sem, 2)

        def rows(c, s):
            return pl.ds(c * CM + s * SUB, SUB)

        def cols(half):
            return pl.ds(half * HL, HL)

        def blk(c, half, s):
            out_ref[rows(c, s), cols(half)] = jnp.dot(
                a_ref[rows(c, s), :], b_ref[:, cols(half)],
                preferred_element_type=jnp.float32,
            )

        def cw_sc(h):
            return lax.rem(my + 3 - h, N_DEV)

        def cw_rc(h):
            return lax.rem(my + N_DEV + 2 - h, N_DEV)

        def ccw_sc(h):
            return lax.rem(my + 1 + h, N_DEV)

        def ccw_rc(h):
            return lax.rem(my + 2 + h, N_DEV)

        all_descs = []

        def rs_start(h, s, direction):
            if direction == 0:
                d = pltpu.make_async_remote_copy(
                    src_ref=out_ref.at[rows(cw_sc(h), s), cols(0)],
                    dst_ref=cw_comm.at[h, pl.ds(s * SUB, SUB), :],
                    send_sem=cw_ssem.at[2 * h + s],
                    recv_sem=cw_rsem.at[2 * h + s],
                    device_id=(right,), device_id_type=pl.DeviceIdType.MESH,
                )
            else:
                d = pltpu.make_async_remote_copy(
                    src_ref=out_ref.at[rows(ccw_sc(h), s), cols(1)],
                    dst_ref=ccw_comm.at[h, pl.ds(s * SUB, SUB), :],
                    send_sem=ccw_ssem.at[2 * h + s],
                    recv_sem=ccw_rsem.at[2 * h + s],
                    device_id=(left,), device_id_type=pl.DeviceIdType.MESH,
                )
            d.start()
            all_descs.append(d)
            return d

        def ag_start(h, s, direction):
            if direction == 0:
                c = lax.rem(my + N_DEV - h, N_DEV)
                d = pltpu.make_async_remote_copy(
                    src_ref=out_ref.at[rows(c, s), cols(0)],
                    dst_ref=out_ref.at[rows(c, s), cols(0)],
                    send_sem=cw_ssem.at[6 + 2 * h + s],
                    recv_sem=cw_rsem.at[6 + 2 * h + s],
                    device_id=(right,), device_id_type=pl.DeviceIdType.MESH,
                )
            else:
                c = lax.rem(my + h, N_DEV)
                d = pltpu.make_async_remote_copy(
                    src_ref=out_ref.at[rows(c, s), cols(1)],
                    dst_ref=out_ref.at[rows(c, s), cols(1)],
                    send_sem=ccw_ssem.at[6 + 2 * h + s],
                    recv_sem=ccw_rsem.at[6 + 2 * h + s],
                    device_id=(left,), device_id_type=pl.DeviceIdType.MESH,
                )
            d.start()
            all_descs.append(d)
            return d

        rs_inflight = {}
        for s in range(NSUB):
            blk(cw_sc(0), 0, s)
            cw_d = rs_start(0, s, 0)
            blk(ccw_sc(0), 1, s)
            ccw_d = rs_start(0, s, 1)
            rs_inflight[s] = (cw_d, ccw_d)

        ag_inflight = {}
        for h in range(N_DEV - 1):
            for s in range(NSUB):
                blk(cw_rc(h), 0, s)
                blk(ccw_rc(h), 1, s)
                cw_d, ccw_d = rs_inflight[s]
                cw_d.wait_recv()
                out_ref[rows(cw_rc(h), s), cols(0)] = (
                    out_ref[rows(cw_rc(h), s), cols(0)]
                    + cw_comm[h, pl.ds(s * SUB, SUB), :]
                )
                if h < N_DEV - 2:
                    new_cw = rs_start(h + 1, s, 0)
                ccw_d.wait_recv()
                out_ref[rows(ccw_rc(h), s), cols(1)] = (
                    out_ref[rows(ccw_rc(h), s), cols(1)]
                    + ccw_comm[h, pl.ds(s * SUB, SUB), :]
                )
                if h < N_DEV - 2:
                    rs_inflight[s] = (new_cw, rs_start(h + 1, s, 1))
                else:
                    out_ref[rows(my, s), :] = jnp.maximum(
                        out_ref[rows(my, s), :], 0.0
                    )
                    ag_inflight[s] = (ag_start(0, s, 0), ag_start(0, s, 1))

        for h in range(N_DEV - 1):
            for s in range(NSUB):
                cw_d, ccw_d = ag_inflight[s]
                cw_d.wait_recv()
                if h < N_DEV - 2:
                    new_cw = ag_start(h + 1, s, 0)
                ccw_d.wait_recv()
                if h < N_DEV - 2:
                    ag_inflight[s] = (new_cw, ag_start(h + 1, s, 1))

        for d in all_descs:
            d.wait_send()

    n_sems = 2 * (N_DEV - 1) * NSUB
    return pl.pallas_call(
        body,
        out_shape=jax.ShapeDtypeStruct((m, n), jnp.float32),
        in_specs=[
            pl.BlockSpec(memory_space=pltpu.VMEM),
            pl.BlockSpec(memory_space=pltpu.VMEM),
        ],
        out_specs=pl.BlockSpec(memory_space=pltpu.VMEM),
        scratch_shapes=[
            pltpu.VMEM((N_DEV - 1, CM, HL), jnp.float32),
            pltpu.VMEM((N_DEV - 1, CM, HL), jnp.float32),
            pltpu.SemaphoreType.DMA((n_sems,)),
            pltpu.SemaphoreType.DMA((n_sems,)),
            pltpu.SemaphoreType.DMA((n_sems,)),
            pltpu.SemaphoreType.DMA((n_sems,)),
        ],
        compiler_params=pltpu.CompilerParams(collective_id=0),
    )(A, B)


# device time: 98407 ns/iter; 3.2405x vs baseline; 1.6987x over previous
import jax
import jax.numpy as jnp
from jax import lax
from jax.experimental import pallas as pl
from jax.experimental.pallas import tpu as pltpu

N_DEV = 4
CM = 512
HL = 1024
NSUB = 2
SUB = CM // NSUB


def kernel(A, B):
    m, k = A.shape
    _, n = B.shape

    def body(a_ref, b_ref, out_ref,
             cw_comm, ccw_comm, cw_stage, ccw_stage, ag_cw, ag_ccw,
             cw_ssem, cw_rsem, ccw_ssem, ccw_rsem):
        my = lax.axis_index("i")
        left = lax.rem(my + N_DEV - 1, N_DEV)
        right = lax.rem(my + 1, N_DEV)

        barrier_sem = pltpu.get_barrier_semaphore()
        for nbr in (left, right):
            pl.semaphore_signal(
                barrier_sem, inc=1,
                device_id=(nbr,), device_id_type=pl.DeviceIdType.MESH,
            )
        pl.semaphore_wait(barrier_sem, 2)

        def rows(c, s):
            return pl.ds(c * CM + s * SUB, SUB)

        def srows(s):
            return pl.ds(s * SUB, SUB)

        def cols(half):
            return pl.ds(half * HL, HL)

        def cw_sc(h):
            return lax.rem(my + 3 - h, N_DEV)

        def cw_rc(h):
            return lax.rem(my + N_DEV + 2 - h, N_DEV)

        def ccw_sc(h):
            return lax.rem(my + 1 + h, N_DEV)

        def ccw_rc(h):
            return lax.rem(my + 2 + h, N_DEV)

        all_descs = []

        def rs_start(h, s, direction):
            stage, comm = (cw_stage, cw_comm) if direction == 0 else (
                ccw_stage, ccw_comm)
            ssem, rsem = (cw_ssem, cw_rsem) if direction == 0 else (
                ccw_ssem, ccw_rsem)
            tgt = right if direction == 0 else left
            d = pltpu.make_async_remote_copy(
                src_ref=stage.at[h, srows(s), :],
                dst_ref=comm.at[h, srows(s), :],
                send_sem=ssem.at[2 * h + s],
                recv_sem=rsem.at[2 * h + s],
                device_id=(tgt,), device_id_type=pl.DeviceIdType.MESH,
            )
            d.start()
            all_descs.append(d)
            return d

        def ag_start(h, s, direction):
            buf = ag_cw if direction == 0 else ag_ccw
            ssem, rsem = (cw_ssem, cw_rsem) if direction == 0 else (
                ccw_ssem, ccw_rsem)
            tgt = right if direction == 0 else left
            d = pltpu.make_async_remote_copy(
                src_ref=buf.at[h, srows(s), :],
                dst_ref=buf.at[h + 1, srows(s), :],
                send_sem=ssem.at[6 + 2 * h + s],
                recv_sem=rsem.at[6 + 2 * h + s],
                device_id=(tgt,), device_id_type=pl.DeviceIdType.MESH,
            )
            d.start()
            all_descs.append(d)
            return d

        rs_inflight = {}
        for s in range(NSUB):
            v = jnp.dot(a_ref[rows(cw_sc(0), s), :], b_ref[:, cols(0)],
                        preferred_element_type=jnp.float32)
            out_ref[rows(cw_sc(0), s), cols(0)] = v
            cw_stage[0, srows(s), :] = v.astype(jnp.bfloat16)
            cw_d = rs_start(0, s, 0)
            v = jnp.dot(a_ref[rows(ccw_sc(0), s), :], b_ref[:, cols(1)],
                        preferred_element_type=jnp.float32)
            out_ref[rows(ccw_sc(0), s), cols(1)] = v
            ccw_stage[0, srows(s), :] = v.astype(jnp.bfloat16)
            ccw_d = rs_start(0, s, 1)
            rs_inflight[s] = (cw_d, ccw_d)

        ag_inflight = {}
        for h in range(N_DEV - 1):
            for s in range(NSUB):
                out_ref[rows(cw_rc(h), s), cols(0)] = jnp.dot(
                    a_ref[rows(cw_rc(h), s), :], b_ref[:, cols(0)],
                    preferred_element_type=jnp.float32)
                out_ref[rows(ccw_rc(h), s), cols(1)] = jnp.dot(
                    a_ref[rows(ccw_rc(h), s), :], b_ref[:, cols(1)],
                    preferred_element_type=jnp.float32)

                cw_d, ccw_d = rs_inflight[s]
                cw_d.wait_recv()
                acc = (out_ref[rows(cw_rc(h), s), cols(0)]
                       + cw_comm[h, srows(s), :].astype(jnp.float32))
                out_ref[rows(cw_rc(h), s), cols(0)] = acc
                if h < N_DEV - 2:
                    cw_stage[h + 1, srows(s), :] = acc.astype(jnp.bfloat16)
                    new_cw = rs_start(h + 1, s, 0)

                ccw_d.wait_recv()
                acc = (out_ref[rows(ccw_rc(h), s), cols(1)]
                       + ccw_comm[h, srows(s), :].astype(jnp.float32))
                out_ref[rows(ccw_rc(h), s), cols(1)] = acc
                if h < N_DEV - 2:
                    ccw_stage[h + 1, srows(s), :] = acc.astype(jnp.bfloat16)
                    rs_inflight[s] = (new_cw, rs_start(h + 1, s, 1))
                else:
                    r = jnp.maximum(out_ref[rows(my, s), :], 0.0)
                    out_ref[rows(my, s), :] = r
                    ag_cw[0, srows(s), :] = r[:, :HL].astype(jnp.bfloat16)
                    ag_ccw[0, srows(s), :] = r[:, HL:].astype(jnp.bfloat16)
                    ag_inflight[s] = (ag_start(0, s, 0), ag_start(0, s, 1))

        for h in range(N_DEV - 1):
            for s in range(NSUB):
                cw_d, ccw_d = ag_inflight[s]
                cw_d.wait_recv()
                if h < N_DEV - 2:
                    new_cw = ag_start(h + 1, s, 0)
                out_ref[rows(lax.rem(my + 3 - h, N_DEV), s), cols(0)] = (
                    ag_cw[h + 1, srows(s), :].astype(jnp.float32))
                ccw_d.wait_recv()
                if h < N_DEV - 2:
                    ag_inflight[s] = (new_cw, ag_start(h + 1, s, 1))
                out_ref[rows(lax.rem(my + 1 + h, N_DEV), s), cols(1)] = (
                    ag_ccw[h + 1, srows(s), :].astype(jnp.float32))

        for d in all_descs:
            d.wait_send()

    n_sems = 2 * (N_DEV - 1) * NSUB
    bf = jnp.bfloat16
    return pl.pallas_call(
        body,
        out_shape=jax.ShapeDtypeStruct((m, n), jnp.float32),
        in_specs=[
            pl.BlockSpec(memory_space=pltpu.VMEM),
            pl.BlockSpec(memory_space=pltpu.VMEM),
        ],
        out_specs=pl.BlockSpec(memory_space=pltpu.VMEM),
        scratch_shapes=[
            pltpu.VMEM((N_DEV - 1, CM, HL), bf),
            pltpu.VMEM((N_DEV - 1, CM, HL), bf),
            pltpu.VMEM((N_DEV - 1, CM, HL), bf),
            pltpu.VMEM((N_DEV - 1, CM, HL), bf),
            pltpu.VMEM((N_DEV, CM, HL), bf),
            pltpu.VMEM((N_DEV, CM, HL), bf),
            pltpu.SemaphoreType.DMA((n_sems,)),
            pltpu.SemaphoreType.DMA((n_sems,)),
            pltpu.SemaphoreType.DMA((n_sems,)),
            pltpu.SemaphoreType.DMA((n_sems,)),
        ],
        compiler_params=pltpu.CompilerParams(
            collective_id=0, vmem_limit_bytes=100 * 1024 * 1024,
        ),
    )(A, B)
